# baseline (speedup 1.0000x reference)
import numpy as np
import jax
import jax.numpy as jnp
from jax import lax
from jax.experimental import pallas as pl
from jax.experimental.pallas import tpu as pltpu
from jax.experimental.pallas import tpu_sc as plsc

_N = 512

_rng = np.random.default_rng(0)
_off = _rng.integers(-2, 0, size=2)
_CY = _N // 2 + int(_off[0])
_CX = _N // 2 + int(_off[1])
_MAXR = int((_N // 2 - 1) * 0.6)
_MINR = int((_N // 2 - 1) * 0.1)
_L = int(_rng.integers(_MINR, _MAXR))
_S = int(_rng.integers(0, _L))
_L2 = _L * _L
_S2 = _S * _S

_R0 = ((_CY - _L + 1) // 8) * 8
_R1 = -((-(_CY + _L)) // 8) * 8
_C0 = ((_CX - _L + 1) // 128) * 128
_C1 = -((-(_CX + _L)) // 128) * 128
_BBR = _R1 - _R0
_BBC = _C1 - _C0

_NIMG = 96
_NW = 32
_IPW = _NIMG // _NW

_ZR = 96
_LANES = 16


def _sc_body(x_hbm, out_hbm, zero_v, band_v, cd_v, zsem):
    wid = lax.axis_index("s") * 2 + lax.axis_index("c")

    zv = jnp.zeros((_LANES,), jnp.float32)
    for k in range(_BBC // _LANES):
        c = _C0 + k * _LANES + lax.iota(jnp.int32, _LANES)
        cd = c - _CX
        cd_v[pl.ds(k * _LANES, _LANES)] = cd * cd

    def _zrow(i, carry):
        for k in range(_N // _LANES):
            zero_v[i, pl.ds(k * _LANES, _LANES)] = zv
        return carry

    lax.fori_loop(0, _ZR, _zrow, 0)

    for t in range(_IPW):
        img = wid * _IPW + t
        zcopies = [
            pltpu.async_copy(zero_v.at[:, :], out_hbm.at[img, pl.ds(0, _ZR), :], zsem),
            pltpu.async_copy(zero_v.at[pl.ds(0, _R0 - _ZR), :],
                             out_hbm.at[img, pl.ds(_ZR, _R0 - _ZR), :], zsem),
            pltpu.async_copy(zero_v.at[:, :], out_hbm.at[img, pl.ds(_R1, _ZR), :], zsem),
            pltpu.async_copy(zero_v.at[pl.ds(0, _N - _R1 - _ZR), :],
                             out_hbm.at[img, pl.ds(_R1 + _ZR, _N - _R1 - _ZR), :], zsem),
            pltpu.async_copy(zero_v.at[:, pl.ds(0, _C0)],
                             out_hbm.at[img, pl.ds(_R0, _ZR), pl.ds(0, _C0)], zsem),
            pltpu.async_copy(zero_v.at[pl.ds(0, _BBR - _ZR), pl.ds(0, _C0)],
                             out_hbm.at[img, pl.ds(_R0 + _ZR, _BBR - _ZR), pl.ds(0, _C0)], zsem),
            pltpu.async_copy(zero_v.at[:, pl.ds(0, _N - _C1)],
                             out_hbm.at[img, pl.ds(_R0, _ZR), pl.ds(_C1, _N - _C1)], zsem),
            pltpu.async_copy(zero_v.at[pl.ds(0, _BBR - _ZR), pl.ds(0, _N - _C1)],
                             out_hbm.at[img, pl.ds(_R0 + _ZR, _BBR - _ZR), pl.ds(_C1, _N - _C1)], zsem),
        ]

        pltpu.sync_copy(x_hbm.at[img, pl.ds(_R0, _BBR), pl.ds(_C0, _BBC)], band_v)

        def _row(i, carry):
            dr = _R0 + i - _CY
            dr2 = dr * dr
            for k in range(_BBC // _LANES):
                sl = pl.ds(k * _LANES, _LANES)
                d2 = dr2 + cd_v[sl]
                m = (d2 < _L2) & (d2 >= _S2)
                band_v[i, sl] = jnp.where(m, band_v[i, sl], 0.0)
            return carry

        lax.fori_loop(0, _BBR, _row, 0)

        pltpu.sync_copy(band_v, out_hbm.at[img, pl.ds(_R0, _BBR), pl.ds(_C0, _BBC)])
        for h in zcopies:
            h.wait()


def _make_sc_call(interpret=False):
    return pl.kernel(
        _sc_body,
        out_type=jax.ShapeDtypeStruct((_NIMG, _N, _N), jnp.float32),
        mesh=plsc.VectorSubcoreMesh(core_axis_name="c", subcore_axis_name="s",
                                    num_cores=2, num_subcores=16),
        scratch_types=[
            pltpu.VMEM((_ZR, _N), jnp.float32),
            pltpu.VMEM((_BBR, _BBC), jnp.float32),
            pltpu.VMEM((_BBC,), jnp.int32),
            pltpu.SemaphoreType.DMA,
        ],
        interpret=interpret,
    )


_sc_call = _make_sc_call()


def kernel(x):
    xr = x.reshape(_NIMG, _N, _N)
    return _sc_call(xr).reshape(x.shape)


# speedup vs baseline: 2.9021x; 2.9021x over previous
import numpy as np
import jax
import jax.numpy as jnp
from jax import lax
from jax.experimental import pallas as pl

_N = 512

_rng = np.random.default_rng(0)
_off = _rng.integers(-2, 0, size=2)
_CY = _N // 2 + int(_off[0])
_CX = _N // 2 + int(_off[1])
_MAXR = int((_N // 2 - 1) * 0.6)
_MINR = int((_N // 2 - 1) * 0.1)
_L = int(_rng.integers(_MINR, _MAXR))
_S = int(_rng.integers(0, _L))
_L2 = _L * _L
_S2 = _S * _S

_R0 = ((_CY - _L + 1) // 8) * 8
_R1 = -((-(_CY + _L)) // 8) * 8
_C0 = ((_CX - _L + 1) // 128) * 128
_C1 = -((-(_CX + _L)) // 128) * 128
_BBR = _R1 - _R0
_BBC = _C1 - _C0

_IPB = 8
_NIMG = 96


def _body(in_ref, out_ref):
    r = _R0 + lax.broadcasted_iota(jnp.int32, (_BBR, _BBC), 0)
    c = _C0 + lax.broadcasted_iota(jnp.int32, (_BBR, _BBC), 1)
    d2 = (r - _CY) ** 2 + (c - _CX) ** 2
    m = (d2 < _L2) & (d2 >= _S2)
    out_ref[:, 0:_R0, :] = jnp.zeros((_IPB, _R0, _N), jnp.float32)
    out_ref[:, _R1:_N, :] = jnp.zeros((_IPB, _N - _R1, _N), jnp.float32)
    out_ref[:, _R0:_R1, 0:_C0] = jnp.zeros((_IPB, _BBR, _C0), jnp.float32)
    out_ref[:, _R0:_R1, _C1:_N] = jnp.zeros((_IPB, _BBR, _N - _C1), jnp.float32)
    out_ref[:, _R0:_R1, _C0:_C1] = jnp.where(m[None], in_ref[...], 0.0)


def _make_call(interpret=False):
    return pl.pallas_call(
        _body,
        grid=(_NIMG // _IPB,),
        in_specs=[pl.BlockSpec(
            (pl.Element(_IPB), pl.Element(_BBR), pl.Element(_BBC)),
            lambda i: (i * _IPB, _R0, _C0),
        )],
        out_specs=pl.BlockSpec((_IPB, _N, _N), lambda i: (i, 0, 0)),
        out_shape=jax.ShapeDtypeStruct((_NIMG, _N, _N), jnp.float32),
        interpret=interpret,
    )


def kernel(x):
    xr = x.reshape(_NIMG, _N, _N)
    y = _make_call()(xr)
    return y.reshape(x.shape)
